# Initial kernel scaffold; baseline (speedup 1.0000x reference)
import jax
import jax.numpy as jnp
from jax.experimental import pallas as pl


def kernel(input_, weight):
    raise NotImplementedError("write your pallas kernel here")


# speedup vs baseline: 1.7994x; 1.7994x over previous
import functools

import jax
import jax.numpy as jnp
from jax import lax
from jax.experimental import pallas as pl
from jax.experimental.pallas import tpu as pltpu
from jax.experimental.pallas import tpu_sc as plsc

NUM_EMBEDDINGS = 1000000
EMBED_DIM = 64
BATCH, SEQ = 16384, 50
TOTAL = BATCH * SEQ

NC, NS = 2, 16
NW = NC * NS
PER_W = TOTAL // NW
CHUNK = 512
NCHUNK = PER_W // CHUNK


def _gather_body(idx_hbm, tbl_hbm, out_hbm, idx_v, rows_v, sem):
    wid = lax.axis_index("s") * NC + lax.axis_index("c")
    base = wid * PER_W

    def step(c, _):
        off = base + c * CHUNK
        pltpu.sync_copy(idx_hbm.at[pl.ds(off, CHUNK)], idx_v)
        pltpu.async_copy(tbl_hbm.at[idx_v], rows_v, sem).wait()
        pltpu.sync_copy(rows_v, out_hbm.at[pl.ds(off, CHUNK)])
        return _

    lax.fori_loop(0, NCHUNK, step, 0)


@jax.jit
def _embed(idx_flat, weight):
    k = pl.kernel(
        _gather_body,
        out_type=jax.ShapeDtypeStruct((TOTAL, EMBED_DIM), jnp.float32),
        mesh=plsc.VectorSubcoreMesh(core_axis_name="c", subcore_axis_name="s"),
        scratch_types=[
            pltpu.VMEM((CHUNK,), jnp.int32),
            pltpu.VMEM((CHUNK, EMBED_DIM), jnp.float32),
            pltpu.SemaphoreType.DMA,
        ],
        compiler_params=pltpu.CompilerParams(use_tc_tiling_on_sc=False),
    )
    return k(idx_flat, weight)


def kernel(input_, weight):
    idx_flat = input_.astype(jnp.int32).reshape(TOTAL)
    out = _embed(idx_flat, weight)
    return out.reshape(BATCH, SEQ, EMBED_DIM)


# speedup vs baseline: 1.8763x; 1.0427x over previous
import functools

import jax
import jax.numpy as jnp
from jax import lax
from jax.experimental import pallas as pl
from jax.experimental.pallas import tpu as pltpu
from jax.experimental.pallas import tpu_sc as plsc

NUM_EMBEDDINGS = 1000000
EMBED_DIM = 64
BATCH, SEQ = 16384, 50
TOTAL = BATCH * SEQ

NC, NS = 2, 16
NW = NC * NS
PER_W = TOTAL // NW
CHUNK = 256
NCHUNK = PER_W // CHUNK
NBUF = 4
K = 2


def _gather_body(idx_hbm, tbl_hbm, out_hbm, idx_all, rows_v, gsem, osem):
    wid = lax.axis_index("s") * NC + lax.axis_index("c")
    base = wid * PER_W
    pltpu.sync_copy(idx_hbm.at[pl.ds(base, PER_W)], idx_all)

    def gather_desc(c):
        b = lax.rem(c, NBUF)
        return pltpu.make_async_copy(
            tbl_hbm.at[idx_all.at[pl.ds(c * CHUNK, CHUNK)]],
            rows_v.at[b],
            gsem.at[b],
        )

    def out_desc(c):
        b = lax.rem(c, NBUF)
        return pltpu.make_async_copy(
            rows_v.at[b],
            out_hbm.at[pl.ds(base + c * CHUNK, CHUNK)],
            osem.at[b],
        )

    for c in range(K):
        gather_desc(c).start()

    def step(c, carry):
        gather_desc(c).wait()
        out_desc(c).start()

        @pl.when(c + K >= NBUF)
        def _wait_buf():
            out_desc(c + K - NBUF).wait()

        gather_desc(c + K).start()
        return carry

    lax.fori_loop(0, NCHUNK - K, step, 0)

    for c in range(NCHUNK - K, NCHUNK):
        gather_desc(c).wait()
        out_desc(c).start()
    for c in range(NCHUNK - NBUF, NCHUNK):
        out_desc(c).wait()


@jax.jit
def _embed(idx_flat, weight):
    k = pl.kernel(
        _gather_body,
        out_type=jax.ShapeDtypeStruct((TOTAL, EMBED_DIM), jnp.float32),
        mesh=plsc.VectorSubcoreMesh(core_axis_name="c", subcore_axis_name="s"),
        scratch_types=[
            pltpu.VMEM((PER_W,), jnp.int32),
            pltpu.VMEM((NBUF, CHUNK, EMBED_DIM), jnp.float32),
            pltpu.SemaphoreType.DMA((NBUF,)),
            pltpu.SemaphoreType.DMA((NBUF,)),
        ],
        compiler_params=pltpu.CompilerParams(use_tc_tiling_on_sc=False),
    )
    return k(idx_flat, weight)


def kernel(input_, weight):
    idx_flat = input_.astype(jnp.int32).reshape(TOTAL)
    out = _embed(idx_flat, weight)
    return out.reshape(BATCH, SEQ, EMBED_DIM)
